# Initial kernel scaffold; baseline (speedup 1.0000x reference)
import jax
import jax.numpy as jnp
from jax.experimental import pallas as pl


def kernel(points, features, W_q, b_q, W_k, b_k, W_v, b_v, W_p1, g_p, be_p, W_p2, b_p2, g_w1, be_w1, W_w1, g_w2, be_w2, W_w2, b_w2):
    raise NotImplementedError("write your pallas kernel here")


# speedup vs baseline: 6.1279x; 6.1279x over previous
import functools

import jax
import jax.numpy as jnp
from jax.experimental import pallas as pl
from jax.experimental.pallas import tpu as pltpu

_R = 8
_OFFS = tuple(list(range(-_R, 0)) + list(range(1, _R + 1)))
_EPS = 1e-5


def _body(B, N, BLK, NB, NPB,
          pe, fe, Wq, bq, Wk, bk, Wv, bv, Wp1, gp, bep, Wp2, bp2, Wp2s, bp2s,
          g1, be1, Ww1, g2, be2, Ww2, bw2, out, sp, sw, sh, wbuf):
    S = 2 * _R
    E = BLK + 2 * _R
    NE = N + 2 * _R
    CNT = float(B * N * S)
    g = pl.program_id(0)

    @pl.when(g == 0)
    def _init():
        sw[...] = jnp.zeros_like(sw)
        sh[...] = jnp.zeros_like(sh)
        m00 = jnp.zeros((1, 1), jnp.float32)
        m01 = jnp.zeros((1, 1), jnp.float32)
        m11 = jnp.zeros((1, 1), jnp.float32)
        st0 = jnp.zeros((1, 1), jnp.float32)
        st1 = jnp.zeros((1, 1), jnp.float32)
        for b in range(B):
            base = b * NE
            pc = pe[base + _R:base + _R + N, :]
            sw_acc = jnp.zeros((N, 2), jnp.float32)
            for d in _OFFS:
                sw_acc = sw_acc + pe[base + _R + d:base + _R + d + N, :]
            p0 = pc[:, 0:1]
            p1 = pc[:, 1:2]
            s0 = sw_acc[:, 0:1]
            s1 = sw_acc[:, 1:2]
            g00 = jnp.sum(p0 * p0)
            g01 = jnp.sum(p0 * p1)
            g11 = jnp.sum(p1 * p1)
            a00 = jnp.sum(s0 * p0)
            a01 = jnp.sum(s0 * p1)
            a10 = jnp.sum(s1 * p0)
            a11 = jnp.sum(s1 * p1)
            m00 = m00 + 2.0 * S * g00 - 2.0 * a00
            m01 = m01 + 2.0 * S * g01 - (a01 + a10)
            m11 = m11 + 2.0 * S * g11 - 2.0 * a11
            st0 = st0 + jnp.sum(s0) - S * jnp.sum(p0)
            st1 = st1 + jnp.sum(s1) - S * jnp.sum(p1)
        w0 = Wp1[0:1, :]
        w1 = Wp1[1:2, :]
        sp[0:1, :] = st0 * w0 + st1 * w1
        sp[1:2, :] = w0 * w0 * m00 + 2.0 * w0 * w1 * m01 + w1 * w1 * m11

    def bn_eff(stat_ref, gamma, beta):
        mean = stat_ref[0:1, :] / CNT
        var = stat_ref[1:2, :] / CNT - mean * mean
        a = gamma * jax.lax.rsqrt(var + _EPS)
        return a, beta - mean * a

    def fill_wbuf(blk):
        b = blk // NPB
        j = blk % NPB
        base = b * NE + j * BLK
        fex = fe[pl.ds(base, E), :]
        pev = pe[pl.ds(base, E), :]
        fc = fex[_R:_R + BLK]
        qv = jnp.dot(fc, Wq[...], preferred_element_type=jnp.float32) + bq[...]
        kx = jnp.dot(fex, Wk[...], preferred_element_type=jnp.float32) + bk[...]
        ap, bp_ = bn_eff(sp, gp[...], bep[...])
        pc = pev[_R:_R + BLK]
        for i, d in enumerate(_OFFS):
            o = _R + d
            t = pev[o:o + BLK] - pc
            r1 = t[:, 0:1] * Wp1[0:1, :] + t[:, 1:2] * Wp1[1:2, :]
            rb = jnp.maximum(r1 * ap + bp_, 0.0)
            rs = (rb[:, 0:1] * Wp2s[0:1, :] + rb[:, 1:2] * Wp2s[1:2, :]
                  + bp2s[...])
            wbuf[i * BLK:(i + 1) * BLK, :] = rs + kx[o:o + BLK] - qv
        return base, fex, pev, pc

    @pl.when((g >= 1) & (g <= NB))
    def _pass_a():
        fill_wbuf(g - 1)
        wcat = wbuf[...]
        sw[0:1, :] += jnp.sum(wcat, axis=0, keepdims=True)
        sw[1:2, :] += jnp.sum(wcat * wcat, axis=0, keepdims=True)

    @pl.when((g > NB) & (g <= 2 * NB))
    def _pass_b():
        fill_wbuf(g - 1 - NB)
        a1, b1 = bn_eff(sw, g1[...], be1[...])
        wn = jnp.maximum(wbuf[...] * a1 + b1, 0.0)
        hcat = jnp.dot(wn, Ww1[...], preferred_element_type=jnp.float32)
        sh[0:1, :] += jnp.sum(hcat, axis=0, keepdims=True)
        sh[1:2, :] += jnp.sum(hcat * hcat, axis=0, keepdims=True)

    @pl.when(g > 2 * NB)
    def _pass_c():
        _, fex, pev, pc = fill_wbuf(g - 1 - 2 * NB)
        a1, b1 = bn_eff(sw, g1[...], be1[...])
        wn = jnp.maximum(wbuf[...] * a1 + b1, 0.0)
        hcat = jnp.dot(wn, Ww1[...], preferred_element_type=jnp.float32)
        a2, b2 = bn_eff(sh, g2[...], be2[...])
        hn = jnp.maximum(hcat * a2 + b2, 0.0)
        acat = jnp.dot(hn, Ww2[...], preferred_element_type=jnp.float32) + bw2[...]
        m = jnp.max(acat, axis=1, keepdims=True)
        e = jnp.exp(acat - m)
        sm = e / jnp.sum(e, axis=1, keepdims=True)
        vx = jnp.dot(fex, Wv[...], preferred_element_type=jnp.float32) + bv[...]
        ap, bp_ = bn_eff(sp, gp[...], bep[...])
        acc = jnp.zeros((BLK, Wv.shape[1]), jnp.float32)
        for i, d in enumerate(_OFFS):
            o = _R + d
            t = pev[o:o + BLK] - pc
            r1 = t[:, 0:1] * Wp1[0:1, :] + t[:, 1:2] * Wp1[1:2, :]
            rb = jnp.maximum(r1 * ap + bp_, 0.0)
            rfull = (rb[:, 0:1] * Wp2[0:1, :] + rb[:, 1:2] * Wp2[1:2, :]
                     + bp2[...])
            vn = vx[o:o + BLK] + rfull
            smi = sm[i * BLK:(i + 1) * BLK]
            w256 = jnp.concatenate([smi] * (Wv.shape[1] // smi.shape[1]), axis=1)
            acc = acc + vn * w256
        out[...] = acc


def kernel(points, features, W_q, b_q, W_k, b_k, W_v, b_v, W_p1, g_p, be_p,
           W_p2, b_p2, g_w1, be_w1, W_w1, g_w2, be_w2, W_w2, b_w2):
    B, N, _ = points.shape
    C = features.shape[1]
    mid = W_q.shape[1]
    out_p = W_v.shape[1]
    BLK = 512
    NPB = N // BLK
    NB = B * NPB

    f3 = features.reshape(B, N, C)
    fe = jnp.concatenate([f3[:, -_R:], f3, f3[:, :_R]], axis=1)
    fe = fe.reshape(B * (N + 2 * _R), C)
    pe = jnp.concatenate([points[:, -_R:], points, points[:, :_R]], axis=1)
    pe = pe.reshape(B * (N + 2 * _R), 2)

    Wp2s = W_p2.reshape(2, out_p // mid, mid).sum(axis=1)
    bp2s = b_p2.reshape(out_p // mid, mid).sum(axis=0)

    def row(x):
        return x.reshape(1, -1)

    operands = (pe, fe, W_q, row(b_q), W_k, row(b_k), W_v, row(b_v),
                W_p1, row(g_p), row(be_p), W_p2, row(b_p2), Wp2s, row(bp2s),
                row(g_w1), row(be_w1), W_w1, row(g_w2), row(be_w2), W_w2,
                row(b_w2))

    grid = (1 + 3 * NB,)
    in_specs = [pl.BlockSpec(x.shape, functools.partial(
        lambda nd, i: (0,) * nd, x.ndim)) for x in operands]
    out_spec = pl.BlockSpec((BLK, out_p),
                            lambda i: (jnp.maximum(i - 1 - 2 * NB, 0), 0))

    body = functools.partial(_body, B, N, BLK, NB, NPB)
    return pl.pallas_call(
        body,
        grid=grid,
        in_specs=in_specs,
        out_specs=out_spec,
        out_shape=jax.ShapeDtypeStruct((B * N, out_p), jnp.float32),
        scratch_shapes=[
            pltpu.VMEM((2, 2), jnp.float32),
            pltpu.VMEM((2, mid), jnp.float32),
            pltpu.VMEM((2, W_w1.shape[1]), jnp.float32),
            pltpu.VMEM((2 * _R * BLK, mid), jnp.float32),
        ],
    )(*operands)


# speedup vs baseline: 8.3487x; 1.3624x over previous
import functools

import jax
import jax.numpy as jnp
from jax.experimental import pallas as pl
from jax.experimental.pallas import tpu as pltpu

_R = 8
_OFFS = tuple(list(range(-_R, 0)) + list(range(1, _R + 1)))
_EPS = 1e-5


def _body(B, N, BLK, NB, NPB,
          peT, fe, Wq, bq, Wk, bk, Wv, bv, Wp1, gp, bep, Wp2, bp2, Wp2s, bp2s,
          g1, be1, Ww1, g2, be2, Ww2, bw2, out,
          sp, sw, sh, wbuf, rb0T, rb1T, rb0C, rb1C):
    S = 2 * _R
    E = BLK + 2 * _R
    NE = N + 2 * _R
    CNT = float(B * N * S)
    g = pl.program_id(0)

    @pl.when(g == 0)
    def _init():
        sw[...] = jnp.zeros_like(sw)
        sh[...] = jnp.zeros_like(sh)
        w00 = Wp1[0:1, 0:1]
        w10 = Wp1[1:2, 0:1]
        w01 = Wp1[0:1, 1:2]
        w11 = Wp1[1:2, 1:2]
        s0 = jnp.zeros((1, 1), jnp.float32)
        s1 = jnp.zeros((1, 1), jnp.float32)
        q0 = jnp.zeros((1, 1), jnp.float32)
        q1 = jnp.zeros((1, 1), jnp.float32)
        for b in range(B):
            p0 = peT[2 * b:2 * b + 1, :]
            p1 = peT[2 * b + 1:2 * b + 2, :]
            p0c = p0[:, _R:_R + N]
            p1c = p1[:, _R:_R + N]
            for i, d in enumerate(_OFFS):
                t0 = p0[:, _R + d:_R + d + N] - p0c
                t1 = p1[:, _R + d:_R + d + N] - p1c
                r10 = t0 * w00 + t1 * w10
                r11 = t0 * w01 + t1 * w11
                rb0T[i:i + 1, b * N:(b + 1) * N] = r10
                rb1T[i:i + 1, b * N:(b + 1) * N] = r11
                s0 = s0 + jnp.sum(r10, keepdims=True).reshape(1, 1)
                s1 = s1 + jnp.sum(r11, keepdims=True).reshape(1, 1)
                q0 = q0 + jnp.sum(r10 * r10, keepdims=True).reshape(1, 1)
                q1 = q1 + jnp.sum(r11 * r11, keepdims=True).reshape(1, 1)
        sp[0:1, 0:1] = s0
        sp[0:1, 1:2] = s1
        sp[1:2, 0:1] = q0
        sp[1:2, 1:2] = q1
        mean = sp[0:1, :] / CNT
        var = sp[1:2, :] / CNT - mean * mean
        a = gp[...] * jax.lax.rsqrt(var + _EPS)
        b_ = bep[...] - mean * a
        rb0C[...] = jnp.transpose(
            jnp.maximum(rb0T[...] * a[0:1, 0:1] + b_[0:1, 0:1], 0.0))
        rb1C[...] = jnp.transpose(
            jnp.maximum(rb1T[...] * a[0:1, 1:2] + b_[0:1, 1:2], 0.0))

    def bn_eff(stat_ref, gamma, beta):
        mean = stat_ref[0:1, :] / CNT
        var = stat_ref[1:2, :] / CNT - mean * mean
        a = gamma * jax.lax.rsqrt(var + _EPS)
        return a, beta - mean * a

    def fill_wbuf(blk):
        b = blk // NPB
        j = blk % NPB
        base = b * NE + j * BLK
        rowb = blk * BLK
        fex = fe[pl.ds(base, E), :]
        fc = fex[_R:_R + BLK]
        qv = jnp.dot(fc, Wq[...], preferred_element_type=jnp.float32) + bq[...]
        kx = jnp.dot(fex, Wk[...], preferred_element_type=jnp.float32) + bk[...]
        for i, d in enumerate(_OFFS):
            o = _R + d
            c0 = rb0C[pl.ds(rowb, BLK), i:i + 1]
            c1 = rb1C[pl.ds(rowb, BLK), i:i + 1]
            rs = c0 * Wp2s[0:1, :] + c1 * Wp2s[1:2, :]
            wbuf[i * BLK:(i + 1) * BLK, :] = (rs + kx[o:o + BLK]
                                              - qv + bp2s[...])
        return rowb, fex

    @pl.when((g >= 1) & (g <= NB))
    def _pass_a():
        fill_wbuf(g - 1)
        wcat = wbuf[...]
        sw[0:1, :] += jnp.sum(wcat, axis=0, keepdims=True)
        sw[1:2, :] += jnp.sum(wcat * wcat, axis=0, keepdims=True)

    @pl.when((g > NB) & (g <= 2 * NB))
    def _pass_b():
        fill_wbuf(g - 1 - NB)
        a1, b1 = bn_eff(sw, g1[...], be1[...])
        wn = jnp.maximum(wbuf[...] * a1 + b1, 0.0)
        hcat = jnp.dot(wn, Ww1[...], preferred_element_type=jnp.float32)
        sh[0:1, :] += jnp.sum(hcat, axis=0, keepdims=True)
        sh[1:2, :] += jnp.sum(hcat * hcat, axis=0, keepdims=True)

    @pl.when(g > 2 * NB)
    def _pass_c():
        rowb, fex = fill_wbuf(g - 1 - 2 * NB)
        a1, b1 = bn_eff(sw, g1[...], be1[...])
        wn = jnp.maximum(wbuf[...] * a1 + b1, 0.0)
        hcat = jnp.dot(wn, Ww1[...], preferred_element_type=jnp.float32)
        a2, b2 = bn_eff(sh, g2[...], be2[...])
        hn = jnp.maximum(hcat * a2 + b2, 0.0)
        acat = jnp.dot(hn, Ww2[...], preferred_element_type=jnp.float32) + bw2[...]
        m = jnp.max(acat, axis=1, keepdims=True)
        e = jnp.exp(acat - m)
        sm = e / jnp.sum(e, axis=1, keepdims=True)
        vx = jnp.dot(fex, Wv[...], preferred_element_type=jnp.float32) + bv[...]
        acc = jnp.zeros((BLK, Wv.shape[1]), jnp.float32)
        for i, d in enumerate(_OFFS):
            o = _R + d
            c0 = rb0C[pl.ds(rowb, BLK), i:i + 1]
            c1 = rb1C[pl.ds(rowb, BLK), i:i + 1]
            rfull = c0 * Wp2[0:1, :] + c1 * Wp2[1:2, :] + bp2[...]
            vn = vx[o:o + BLK] + rfull
            smi = sm[i * BLK:(i + 1) * BLK]
            w256 = jnp.concatenate([smi] * (Wv.shape[1] // smi.shape[1]), axis=1)
            acc = acc + vn * w256
        out[...] = acc


def kernel(points, features, W_q, b_q, W_k, b_k, W_v, b_v, W_p1, g_p, be_p,
           W_p2, b_p2, g_w1, be_w1, W_w1, g_w2, be_w2, W_w2, b_w2):
    B, N, _ = points.shape
    C = features.shape[1]
    mid = W_q.shape[1]
    out_p = W_v.shape[1]
    BLK = 512
    NPB = N // BLK
    NB = B * NPB

    f3 = features.reshape(B, N, C)
    fe = jnp.concatenate([f3[:, -_R:], f3, f3[:, :_R]], axis=1)
    fe = fe.reshape(B * (N + 2 * _R), C)
    pext = jnp.concatenate([points[:, -_R:], points, points[:, :_R]], axis=1)
    peT = pext.transpose(0, 2, 1).reshape(2 * B, N + 2 * _R)

    Wp2s = W_p2.reshape(2, out_p // mid, mid).sum(axis=1)
    bp2s = b_p2.reshape(out_p // mid, mid).sum(axis=0)

    def row(x):
        return x.reshape(1, -1)

    operands = (peT, fe, W_q, row(b_q), W_k, row(b_k), W_v, row(b_v),
                W_p1, row(g_p), row(be_p), W_p2, row(b_p2), Wp2s, row(bp2s),
                row(g_w1), row(be_w1), W_w1, row(g_w2), row(be_w2), W_w2,
                row(b_w2))

    grid = (1 + 3 * NB,)
    in_specs = [pl.BlockSpec(x.shape, functools.partial(
        lambda nd, i: (0,) * nd, x.ndim)) for x in operands]
    out_spec = pl.BlockSpec((BLK, out_p),
                            lambda i: (jnp.maximum(i - 1 - 2 * NB, 0), 0))

    body = functools.partial(_body, B, N, BLK, NB, NPB)
    S = 2 * _R
    return pl.pallas_call(
        body,
        grid=grid,
        in_specs=in_specs,
        out_specs=out_spec,
        out_shape=jax.ShapeDtypeStruct((B * N, out_p), jnp.float32),
        scratch_shapes=[
            pltpu.VMEM((2, 2), jnp.float32),
            pltpu.VMEM((2, mid), jnp.float32),
            pltpu.VMEM((2, W_w1.shape[1]), jnp.float32),
            pltpu.VMEM((S * BLK, mid), jnp.float32),
            pltpu.VMEM((S, B * N), jnp.float32),
            pltpu.VMEM((S, B * N), jnp.float32),
            pltpu.VMEM((B * N, S), jnp.float32),
            pltpu.VMEM((B * N, S), jnp.float32),
        ],
    )(*operands)
